# initial kernel scaffold (unmeasured)
import jax
import jax.numpy as jnp
from jax import lax
from jax.experimental import pallas as pl
from jax.experimental.pallas import tpu as pltpu

N_DEV = 32
N_TOK = 1024
D_IN = 512
D_OUT = 1024
N_EXP = 128
EPD = N_EXP // N_DEV
CAP = 6
ROWS_PER = N_TOK // N_DEV
SLOTS = 8


def kernel(x, router_W, route_idx, expert_W):
    del router_W

    e = route_idx[:, 0].astype(jnp.int32)
    tok = jnp.arange(N_TOK, dtype=jnp.int32)
    onehot = (e[:, None] == jnp.arange(N_EXP, dtype=jnp.int32)[None, :])
    csum = jnp.cumsum(onehot.astype(jnp.int32), axis=0)
    rank = jnp.take_along_axis(csum, e[:, None], axis=1)[:, 0] - 1
    kept = rank < CAP
    owner = e // EPD
    slot = jnp.where(kept, (e % EPD) * SLOTS + rank, SLOTS - 1).astype(jnp.int32)
    T = jnp.full((N_EXP, CAP + 1), N_TOK, dtype=jnp.int32)
    T = T.at[e, jnp.minimum(rank, CAP)].set(tok)[:, :CAP]
    dest = tok // ROWS_PER
    self_row = (owner == dest).astype(jnp.int32)
    self_cnt = jnp.zeros((N_DEV,), jnp.int32).at[dest].add(self_row)
    send_n = jnp.zeros((N_DEV,), jnp.int32).at[owner].add(1 - self_row)
    recv_n = ROWS_PER - self_cnt

    def body(owner_ref, slot_ref, t_ref, sendn_ref, recvn_ref,
             x_ref, w_ref, out_ref, xg_ref, comp_ref, send_sem, recv_sem):
        my = lax.axis_index("i")

        xg_ref[...] = jnp.zeros((EPD * SLOTS, D_IN), jnp.float32)
        for le in range(EPD):
            for s in range(CAP):
                t = t_ref[my * EPD + le, s]

                @pl.when(t < N_TOK)
                def _():
                    xg_ref[pl.ds(le * SLOTS + s, 1), :] = x_ref[pl.ds(t, 1), :]

        for le in range(EPD):
            comp_ref[pl.ds(le * SLOTS, SLOTS), :] = jnp.dot(
                xg_ref[pl.ds(le * SLOTS, SLOTS), :], w_ref[le],
                preferred_element_type=jnp.float32,
            )

        def send_one(i, carry):
            @pl.when((owner_ref[i] == my) & (i // ROWS_PER != my))
            def _():
                rdma = pltpu.make_async_remote_copy(
                    src_ref=comp_ref.at[pl.ds(slot_ref[i], 1), :],
                    dst_ref=out_ref.at[pl.ds(i % ROWS_PER, 1), :],
                    send_sem=send_sem,
                    recv_sem=recv_sem,
                    device_id=(i // ROWS_PER,),
                    device_id_type=pl.DeviceIdType.MESH,
                )
                rdma.start()
            return carry

        lax.fori_loop(0, N_TOK, send_one, 0)

        def local_one(i, carry):
            @pl.when(owner_ref[i] == my)
            def _():
                out_ref[pl.ds(i % ROWS_PER, 1), :] = comp_ref[pl.ds(slot_ref[i], 1), :]
            return carry

        lax.fori_loop(my * ROWS_PER, (my + 1) * ROWS_PER, local_one, 0)

        def wait_recv_one(i, carry):
            rdma = pltpu.make_async_remote_copy(
                src_ref=comp_ref.at[pl.ds(0, 1), :],
                dst_ref=out_ref.at[pl.ds(0, 1), :],
                send_sem=send_sem,
                recv_sem=recv_sem,
                device_id=(my,),
                device_id_type=pl.DeviceIdType.MESH,
            )
            rdma.wait_recv()
            return carry

        lax.fori_loop(0, recvn_ref[my], wait_recv_one, 0)

        def wait_send_one(i, carry):
            rdma = pltpu.make_async_remote_copy(
                src_ref=comp_ref.at[pl.ds(0, 1), :],
                dst_ref=out_ref.at[pl.ds(0, 1), :],
                send_sem=send_sem,
                recv_sem=recv_sem,
                device_id=(my,),
                device_id_type=pl.DeviceIdType.MESH,
            )
            rdma.wait_send()
            return carry

        lax.fori_loop(0, sendn_ref[my], wait_send_one, 0)

    return pl.pallas_call(
        body,
        out_shape=jax.ShapeDtypeStruct((ROWS_PER, D_OUT), jnp.float32),
        in_specs=[
            pl.BlockSpec(memory_space=pltpu.SMEM),
            pl.BlockSpec(memory_space=pltpu.SMEM),
            pl.BlockSpec(memory_space=pltpu.SMEM),
            pl.BlockSpec(memory_space=pltpu.SMEM),
            pl.BlockSpec(memory_space=pltpu.SMEM),
            pl.BlockSpec(memory_space=pltpu.VMEM),
            pl.BlockSpec(memory_space=pltpu.VMEM),
        ],
        out_specs=pl.BlockSpec(memory_space=pltpu.VMEM),
        scratch_shapes=[
            pltpu.VMEM((EPD * SLOTS, D_IN), jnp.float32),
            pltpu.VMEM((EPD * SLOTS, D_OUT), jnp.float32),
            pltpu.SemaphoreType.DMA,
            pltpu.SemaphoreType.DMA,
        ],
        compiler_params=pltpu.CompilerParams(collective_id=0),
    )(owner, slot, T, send_n, recv_n, x, expert_W)


# baseline (device time: 137004 ns/iter reference)
import jax
import jax.numpy as jnp
from jax import lax
from jax.experimental import pallas as pl
from jax.experimental.pallas import tpu as pltpu

N_DEV = 32
N_TOK = 1024
D_IN = 512
D_OUT = 1024
N_EXP = 128
EPD = N_EXP // N_DEV
CAP = 6
ROWS_PER = N_TOK // N_DEV
SLOTS = 8


def kernel(x, router_W, route_idx, expert_W):
    del router_W

    e = route_idx[:, 0].astype(jnp.int32)
    tok = jnp.arange(N_TOK, dtype=jnp.int32)
    onehot = (e[:, None] == jnp.arange(N_EXP, dtype=jnp.int32)[None, :])
    csum = jnp.cumsum(onehot.astype(jnp.int32), axis=0)
    rank = jnp.take_along_axis(csum, e[:, None], axis=1)[:, 0] - 1
    kept = rank < CAP
    owner = e // EPD
    slot = jnp.where(kept, (e % EPD) * SLOTS + rank, SLOTS - 1).astype(jnp.int32)
    T = jnp.full((N_EXP, CAP + 1), N_TOK, dtype=jnp.int32)
    T = T.at[e, jnp.minimum(rank, CAP)].set(tok)[:, :CAP]
    dest = tok // ROWS_PER
    self_row = (owner == dest).astype(jnp.int32)
    self_cnt = jnp.zeros((N_DEV,), jnp.int32).at[dest].add(self_row)
    send_n = jnp.zeros((N_DEV,), jnp.int32).at[owner].add(1 - self_row)
    recv_n = ROWS_PER - self_cnt

    def body(owner_ref, slot_ref, t_ref, sendn_ref, recvn_ref,
             x_ref, w_ref, out_ref, xg_ref, comp_ref, send_sem, recv_sem):
        my = lax.axis_index("i")

        xg_ref[...] = jnp.zeros((EPD * SLOTS, D_IN), jnp.float32)
        for le in range(EPD):
            for s in range(CAP):
                t = t_ref[my * EPD + le, s]

                @pl.when(t < N_TOK)
                def _():
                    xg_ref[pl.ds(le * SLOTS + s, 1), :] = x_ref[pl.ds(t, 1), :]

        for le in range(EPD):
            comp_ref[pl.ds(le * SLOTS, SLOTS), :] = jnp.dot(
                xg_ref[pl.ds(le * SLOTS, SLOTS), :], w_ref[le],
                preferred_element_type=jnp.float32,
            )

        def send_one(i, carry):
            @pl.when((owner_ref[i] == my) & (i // ROWS_PER != my))
            def _():
                rdma = pltpu.make_async_remote_copy(
                    src_ref=comp_ref.at[pl.ds(slot_ref[i], 1), :],
                    dst_ref=out_ref.at[pl.ds(i % ROWS_PER, 1), :],
                    send_sem=send_sem,
                    recv_sem=recv_sem,
                    device_id=(i // ROWS_PER,),
                    device_id_type=pl.DeviceIdType.MESH,
                )
                rdma.start()
            return carry

        lax.fori_loop(0, N_TOK, send_one, 0)

        def local_one(i, carry):
            @pl.when(owner_ref[i] == my)
            def _():
                out_ref[pl.ds(i % ROWS_PER, 1), :] = comp_ref[pl.ds(slot_ref[i], 1), :]
            return carry

        lax.fori_loop(my * ROWS_PER, (my + 1) * ROWS_PER, local_one, 0)

        def wait_recv_one(i, carry):
            rdma = pltpu.make_async_remote_copy(
                src_ref=comp_ref.at[pl.ds(0, 1), :],
                dst_ref=out_ref.at[pl.ds(0, 1), :],
                send_sem=send_sem,
                recv_sem=recv_sem,
                device_id=(my,),
                device_id_type=pl.DeviceIdType.MESH,
            )
            rdma.wait_recv()
            return carry

        lax.fori_loop(0, recvn_ref[my], wait_recv_one, 0)

        def wait_send_one(i, carry):
            rdma = pltpu.make_async_remote_copy(
                src_ref=comp_ref.at[pl.ds(0, 1), :],
                dst_ref=out_ref.at[pl.ds(0, 1), :],
                send_sem=send_sem,
                recv_sem=recv_sem,
                device_id=(my,),
                device_id_type=pl.DeviceIdType.MESH,
            )
            rdma.wait_send()
            return carry

        lax.fori_loop(0, sendn_ref[my], wait_send_one, 0)

    return pl.pallas_call(
        body,
        out_shape=jax.ShapeDtypeStruct((ROWS_PER, D_OUT), jnp.float32),
        in_specs=[
            pl.BlockSpec(memory_space=pltpu.SMEM),
            pl.BlockSpec(memory_space=pltpu.SMEM),
            pl.BlockSpec(memory_space=pltpu.SMEM),
            pl.BlockSpec(memory_space=pltpu.SMEM),
            pl.BlockSpec(memory_space=pltpu.SMEM),
            pl.BlockSpec(memory_space=pltpu.VMEM),
            pl.BlockSpec(memory_space=pltpu.VMEM),
        ],
        out_specs=pl.BlockSpec(memory_space=pltpu.VMEM),
        scratch_shapes=[
            pltpu.VMEM((EPD * SLOTS, D_IN), jnp.float32),
            pltpu.VMEM((EPD * SLOTS, D_OUT), jnp.float32),
            pltpu.SemaphoreType.DMA,
            pltpu.SemaphoreType.DMA,
        ],
    )(owner, slot, T, send_n, recv_n, x, expert_W)


# device time: 45529 ns/iter; 3.0092x vs baseline; 3.0092x over previous
import jax
import jax.numpy as jnp
from jax import lax
from jax.experimental import pallas as pl
from jax.experimental.pallas import tpu as pltpu

N_DEV = 32
N_TOK = 1024
D_IN = 512
D_OUT = 1024
N_EXP = 128
EPD = N_EXP // N_DEV
CAP = 6
ROWS_PER = N_TOK // N_DEV
SLOTS = 8


def kernel(x, router_W, route_idx, expert_W):
    del router_W
    ridx = route_idx[:, 0]

    def body(ridx_ref, x_ref, w_ref, out_ref,
             xg_ref, comp_ref, cnt_ref, mytok_ref, myslot_ref, meta_ref,
             send_sem, recv_sem):
        my = lax.axis_index("i")

        def zero_cnt(i, c):
            cnt_ref[i] = 0
            return c

        lax.fori_loop(0, N_EXP, zero_cnt, 0)
        meta_ref[0] = 0
        meta_ref[1] = 0

        def scan(i, c):
            e = ridx_ref[i]
            ce = cnt_ref[e]
            cnt_ref[e] = ce + 1

            @pl.when(e // EPD == my)
            def _():
                j = meta_ref[0]
                mytok_ref[j] = i
                myslot_ref[j] = jnp.where(
                    ce < CAP, (e % EPD) * SLOTS + ce, SLOTS - 1
                )
                meta_ref[0] = j + 1
                meta_ref[1] = meta_ref[1] + jnp.where(
                    i // ROWS_PER == my, 1, 0
                )

            return c

        lax.fori_loop(0, N_TOK, scan, 0)
        n_my = meta_ref[0]
        self_c = meta_ref[1]

        xg_ref[...] = jnp.zeros((EPD * SLOTS, D_IN), jnp.float32)

        def gather(j, c):
            s = myslot_ref[j]
            t = mytok_ref[j]

            @pl.when(s != SLOTS - 1)
            def _():
                xg_ref[pl.ds(s, 1), :] = x_ref[pl.ds(t, 1), :]

            return c

        lax.fori_loop(0, n_my, gather, 0)

        for le in range(EPD):
            comp_ref[pl.ds(le * SLOTS, SLOTS), :] = jnp.dot(
                xg_ref[pl.ds(le * SLOTS, SLOTS), :], w_ref[le],
                preferred_element_type=jnp.float32,
            )

        def send_one(j, c):
            t = mytok_ref[j]
            s = myslot_ref[j]
            dst = t // ROWS_PER
            drow = t % ROWS_PER

            @pl.when(dst == my)
            def _():
                out_ref[pl.ds(drow, 1), :] = comp_ref[pl.ds(s, 1), :]

            @pl.when(dst != my)
            def _():
                rdma = pltpu.make_async_remote_copy(
                    src_ref=comp_ref.at[pl.ds(s, 1), :],
                    dst_ref=out_ref.at[pl.ds(drow, 1), :],
                    send_sem=send_sem,
                    recv_sem=recv_sem,
                    device_id=(dst,),
                    device_id_type=pl.DeviceIdType.MESH,
                )
                rdma.start()

            return c

        lax.fori_loop(0, n_my, send_one, 0)

        def wait_recv_one(i, c):
            rdma = pltpu.make_async_remote_copy(
                src_ref=comp_ref.at[pl.ds(0, 1), :],
                dst_ref=out_ref.at[pl.ds(0, 1), :],
                send_sem=send_sem,
                recv_sem=recv_sem,
                device_id=(my,),
                device_id_type=pl.DeviceIdType.MESH,
            )
            rdma.wait_recv()
            return c

        lax.fori_loop(0, ROWS_PER - self_c, wait_recv_one, 0)

        def wait_send_one(i, c):
            rdma = pltpu.make_async_remote_copy(
                src_ref=comp_ref.at[pl.ds(0, 1), :],
                dst_ref=out_ref.at[pl.ds(0, 1), :],
                send_sem=send_sem,
                recv_sem=recv_sem,
                device_id=(my,),
                device_id_type=pl.DeviceIdType.MESH,
            )
            rdma.wait_send()
            return c

        lax.fori_loop(0, n_my - self_c, wait_send_one, 0)

    return pl.pallas_call(
        body,
        out_shape=jax.ShapeDtypeStruct((ROWS_PER, D_OUT), jnp.float32),
        in_specs=[
            pl.BlockSpec(memory_space=pltpu.SMEM),
            pl.BlockSpec(memory_space=pltpu.VMEM),
            pl.BlockSpec(memory_space=pltpu.VMEM),
        ],
        out_specs=pl.BlockSpec(memory_space=pltpu.VMEM),
        scratch_shapes=[
            pltpu.VMEM((EPD * SLOTS, D_IN), jnp.float32),
            pltpu.VMEM((EPD * SLOTS, D_OUT), jnp.float32),
            pltpu.SMEM((N_EXP,), jnp.int32),
            pltpu.SMEM((N_TOK,), jnp.int32),
            pltpu.SMEM((N_TOK,), jnp.int32),
            pltpu.SMEM((2,), jnp.int32),
            pltpu.SemaphoreType.DMA,
            pltpu.SemaphoreType.DMA,
        ],
    )(ridx, x, expert_W)


# device time: 35330 ns/iter; 3.8778x vs baseline; 1.2887x over previous
import jax
import jax.numpy as jnp
from jax import lax
from jax.experimental import pallas as pl
from jax.experimental.pallas import tpu as pltpu

N_DEV = 32
N_TOK = 1024
D_IN = 512
D_OUT = 1024
N_EXP = 128
EPD = N_EXP // N_DEV
CAP = 6
ROWS_PER = N_TOK // N_DEV
SLOTS = 8


def kernel(x, router_W, route_idx, expert_W):
    del router_W
    ridx = route_idx[:, 0]

    def body(ridx_ref, x_ref, w_ref, out_ref,
             xg_ref, comp_ref, cnt_ref, mytok_ref, myslot_ref, meta_ref,
             send_sem, recv_sem):
        my = lax.axis_index("i")

        barrier = pltpu.get_barrier_semaphore()

        def signal_peer(d, c):
            @pl.when(d != my)
            def _():
                pl.semaphore_signal(
                    barrier, inc=1, device_id=(d,),
                    device_id_type=pl.DeviceIdType.MESH,
                )
            return c

        lax.fori_loop(0, N_DEV, signal_peer, 0)

        def zero_cnt(i, c):
            cnt_ref[i] = 0
            return c

        lax.fori_loop(0, N_EXP, zero_cnt, 0)
        meta_ref[0] = 0
        meta_ref[1] = 0

        def scan(i, c):
            e = ridx_ref[i]
            ce = cnt_ref[e]
            cnt_ref[e] = ce + 1

            @pl.when(e // EPD == my)
            def _():
                j = meta_ref[0]
                mytok_ref[j] = i
                myslot_ref[j] = jnp.where(
                    ce < CAP, (e % EPD) * SLOTS + ce, SLOTS - 1
                )
                meta_ref[0] = j + 1
                meta_ref[1] = meta_ref[1] + jnp.where(
                    i // ROWS_PER == my, 1, 0
                )

            return c

        lax.fori_loop(0, N_TOK, scan, 0, unroll=8)
        n_my = meta_ref[0]
        self_c = meta_ref[1]

        xg_ref[...] = jnp.zeros((EPD * SLOTS, D_IN), jnp.float32)

        def gather(j, c):
            s = myslot_ref[j]
            t = mytok_ref[j]

            @pl.when(s != SLOTS - 1)
            def _():
                xg_ref[pl.ds(s, 1), :] = x_ref[pl.ds(t, 1), :]

            return c

        lax.fori_loop(0, n_my, gather, 0)

        for le in range(EPD):
            comp_ref[pl.ds(le * SLOTS, SLOTS), :] = jnp.dot(
                xg_ref[pl.ds(le * SLOTS, SLOTS), :], w_ref[le],
                preferred_element_type=jnp.float32,
            )

        pl.semaphore_wait(barrier, N_DEV - 1)

        def send_one(j, c):
            t = mytok_ref[j]
            s = myslot_ref[j]
            dst = t // ROWS_PER
            drow = t % ROWS_PER

            @pl.when(dst == my)
            def _():
                out_ref[pl.ds(drow, 1), :] = comp_ref[pl.ds(s, 1), :]

            @pl.when(dst != my)
            def _():
                rdma = pltpu.make_async_remote_copy(
                    src_ref=comp_ref.at[pl.ds(s, 1), :],
                    dst_ref=out_ref.at[pl.ds(drow, 1), :],
                    send_sem=send_sem,
                    recv_sem=recv_sem,
                    device_id=(dst,),
                    device_id_type=pl.DeviceIdType.MESH,
                )
                rdma.start()

            return c

        lax.fori_loop(0, n_my, send_one, 0)

        def wait_recv_one(i, c):
            rdma = pltpu.make_async_remote_copy(
                src_ref=comp_ref.at[pl.ds(0, 1), :],
                dst_ref=out_ref.at[pl.ds(0, 1), :],
                send_sem=send_sem,
                recv_sem=recv_sem,
                device_id=(my,),
                device_id_type=pl.DeviceIdType.MESH,
            )
            rdma.wait_recv()
            return c

        lax.fori_loop(0, ROWS_PER - self_c, wait_recv_one, 0)

        def wait_send_one(i, c):
            rdma = pltpu.make_async_remote_copy(
                src_ref=comp_ref.at[pl.ds(0, 1), :],
                dst_ref=out_ref.at[pl.ds(0, 1), :],
                send_sem=send_sem,
                recv_sem=recv_sem,
                device_id=(my,),
                device_id_type=pl.DeviceIdType.MESH,
            )
            rdma.wait_send()
            return c

        lax.fori_loop(0, n_my - self_c, wait_send_one, 0)

    return pl.pallas_call(
        body,
        out_shape=jax.ShapeDtypeStruct((ROWS_PER, D_OUT), jnp.float32),
        in_specs=[
            pl.BlockSpec(memory_space=pltpu.SMEM),
            pl.BlockSpec(memory_space=pltpu.VMEM),
            pl.BlockSpec(memory_space=pltpu.VMEM),
        ],
        out_specs=pl.BlockSpec(memory_space=pltpu.VMEM),
        scratch_shapes=[
            pltpu.VMEM((EPD * SLOTS, D_IN), jnp.float32),
            pltpu.VMEM((EPD * SLOTS, D_OUT), jnp.float32),
            pltpu.SMEM((N_EXP,), jnp.int32),
            pltpu.SMEM((N_TOK,), jnp.int32),
            pltpu.SMEM((N_TOK,), jnp.int32),
            pltpu.SMEM((2,), jnp.int32),
            pltpu.SemaphoreType.DMA,
            pltpu.SemaphoreType.DMA,
        ],
        compiler_params=pltpu.CompilerParams(collective_id=0),
    )(ridx, x, expert_W)
